# baseline (speedup 1.0000x reference)
import functools

import jax
import jax.numpy as jnp
from jax import lax
from jax.experimental import pallas as pl
from jax.experimental.pallas import tpu as pltpu
from jax.experimental.pallas import tpu_sc as plsc

EMBED_DIM = 64
SCALE = 8.0
LOWEST = 1e-13


@jax.jit
def _emb_lookup(tokens, table2):
    R, S = tokens.shape
    D = EMBED_DIM
    info = plsc.get_sparse_core_info()
    NW = info.num_cores * info.num_subcores
    rows_per_w = R // NW
    n_blocks = rows_per_w // 8

    mesh = plsc.VectorSubcoreMesh(core_axis_name="c", subcore_axis_name="s")

    @functools.partial(
        pl.kernel,
        mesh=mesh,
        out_type=jax.ShapeDtypeStruct((R, S, D), jnp.float32),
        compiler_params=pltpu.CompilerParams(needs_layout_passes=False),
        scratch_types=[
            pltpu.VMEM((8, S), jnp.int32),
            pltpu.VMEM((8, S), jnp.int32),
            pltpu.VMEM((256,), jnp.int32),
            pltpu.VMEM((256,), jnp.int32),
            pltpu.VMEM((S, 2 * D), jnp.float32),
            pltpu.VMEM((S, 2 * D), jnp.float32),
            pltpu.VMEM((S, D), jnp.float32),
            pltpu.SemaphoreType.DMA((2,)),
        ],
    )
    def emb(tok_hbm, table_hbm, out_hbm, raw_e, raw_o, pair0, pair1,
            grows0, grows1, outs_v, gsem):
        wid = lax.axis_index("s") * info.num_cores + lax.axis_index("c")
        base = wid * rows_per_w
        raws = (raw_e, raw_o)
        pairs = (pair0, pair1)
        growss = (grows0, grows1)
        lanes = lax.iota(jnp.int32, 16)
        n_grp = S // 16 + 1

        def stage_slab(b, sbuf):
            pltpu.sync_copy(tok_hbm.at[pl.ds(base + b * 8, 8)], raws[sbuf])

        def stage_row(t, sbuf, buf):
            raw_v = raws[sbuf]
            pair_v = pairs[buf]
            for j in range(n_grp):
                c = min(j * 16, S - 16)
                pair_v[pl.ds(c, 16)] = lax.shift_right_logical(
                    raw_v[t, pl.ds(c, 16)], 1
                )
            pltpu.async_copy(
                table_hbm.at[pair_v.at[pl.ds(0, 128)]],
                growss[buf].at[pl.ds(0, 128)],
                gsem.at[buf],
            )
            pltpu.async_copy(
                table_hbm.at[pair_v.at[pl.ds(128, S - 128)]],
                growss[buf].at[pl.ds(128, S - 128)],
                gsem.at[buf],
            )

        def process_row(g, sbuf, t, buf):
            pltpu.make_async_copy(
                table_hbm.at[pl.ds(0, S)],
                growss[buf],
                gsem.at[buf],
            ).wait()
            raw_v = raws[sbuf]
            grows_v = growss[buf]

            def window(c):
                row16 = c + lanes
                h64 = (raw_v[t, pl.ds(c, 16)] & 1) * D

                def estep(e4, _):
                    for u in range(4):
                        e = e4 * 4 + u
                        v = plsc.load_gather(grows_v, [row16, h64 + e])
                        plsc.store_scatter(
                            outs_v,
                            [row16, jnp.full((16,), 0, jnp.int32) + e],
                            v * SCALE + LOWEST,
                        )
                    return 0

                lax.fori_loop(0, D // 4, estep, 0)

            def select_scale(gr, _):
                window(gr * 16)
                return 0

            lax.fori_loop(0, S // 16, select_scale, 0)
            window(S - 16)
            pltpu.sync_copy(outs_v.at[...], out_hbm.at[base + g])

        stage_slab(0, 0)
        stage_row(0, 0, 0)

        def half_block(b, sbuf):
            @pl.when(b + 1 < n_blocks)
            def _():
                stage_slab(b + 1, (sbuf + 1) % 2)

            for t in range(8):
                g = b * 8 + t
                buf = t % 2
                nt = t + 1

                @pl.when(g + 1 < rows_per_w)
                def _(nt=nt, sbuf=sbuf):
                    if nt < 8:
                        stage_row(nt, sbuf, nt % 2)
                    else:
                        stage_row(0, (sbuf + 1) % 2, 0)

                process_row(g, sbuf, t, buf)

        def pair_body(i, _):
            half_block(i * 2, 0)
            half_block(i * 2 + 1, 1)
            return 0

        lax.fori_loop(0, n_blocks // 2, pair_body, 0)

    return emb(tokens, table2)


def kernel(token_tensor, table):
    V, D = table.shape
    table2 = table.reshape(V // 2, 2 * D)
    return _emb_lookup(token_tensor, table2)


# speedup vs baseline: 2.5912x; 2.5912x over previous
import functools

import jax
import jax.numpy as jnp
from jax import lax
from jax.experimental import pallas as pl
from jax.experimental.pallas import tpu as pltpu
from jax.experimental.pallas import tpu_sc as plsc

EMBED_DIM = 64
SCALE = 8.0
LOWEST = 1e-13
T = 4


@jax.jit
def _emb_lookup(tokens, table):
    R, S = tokens.shape
    D = table.shape[1]
    info = plsc.get_sparse_core_info()
    NW = info.num_cores * info.num_subcores
    rows_per_w = R // NW
    n_chunks = rows_per_w // T
    assert n_chunks * T == rows_per_w and n_chunks % 2 == 0

    mesh = plsc.VectorSubcoreMesh(core_axis_name="c", subcore_axis_name="s")

    @functools.partial(
        pl.kernel,
        mesh=mesh,
        out_type=jax.ShapeDtypeStruct((R, S, D), jnp.float32),
        compiler_params=pltpu.CompilerParams(use_tc_tiling_on_sc=False),
        scratch_types=[
            pltpu.VMEM((2, T, S), jnp.int32),
            pltpu.VMEM((2, T, S, D), jnp.float32),
            pltpu.SemaphoreType.DMA((2,)),
        ],
    )
    def emb(tok_hbm, table_hbm, out_hbm, idx_v, rows_v, gsem):
        wid = lax.axis_index("s") * info.num_cores + lax.axis_index("c")
        base = wid * rows_per_w

        def stage(g, buf):
            row0 = base + g * T
            pltpu.sync_copy(tok_hbm.at[pl.ds(row0, T)], idx_v.at[buf])
            for t in range(T):
                pltpu.async_copy(
                    table_hbm.at[idx_v.at[buf, t, pl.ds(0, 128)]],
                    rows_v.at[buf, t, pl.ds(0, 128)],
                    gsem.at[buf],
                )
                pltpu.async_copy(
                    table_hbm.at[idx_v.at[buf, t, pl.ds(128, S - 128)]],
                    rows_v.at[buf, t, pl.ds(128, S - 128)],
                    gsem.at[buf],
                )

        def process(g, buf):
            pltpu.make_async_copy(
                out_hbm.at[pl.ds(0, T)],
                rows_v.at[buf],
                gsem.at[buf],
            ).wait()

            for t in range(T):

                def scale4(r4, _, t=t):
                    r = r4 * 4
                    for u in range(4):
                        for j in range(D // 16):
                            sl = pl.ds(j * 16, 16)
                            rows_v[buf, t, r + u, sl] = (
                                rows_v[buf, t, r + u, sl] * SCALE + LOWEST
                            )
                    return 0

                lax.fori_loop(0, S // 4, scale4, 0)
            pltpu.sync_copy(
                rows_v.at[buf], out_hbm.at[pl.ds(base + g * T, T)]
            )

        stage(0, 0)

        def pair_body(i, _):
            g0 = i * 2
            stage(g0 + 1, 1)
            process(g0, 0)

            @pl.when(g0 + 2 < n_chunks)
            def _():
                stage(g0 + 2, 0)

            process(g0 + 1, 1)
            return 0

        lax.fori_loop(0, n_chunks // 2, pair_body, 0)

    return emb(tokens, table)


def kernel(token_tensor, table):
    return _emb_lookup(token_tensor, table)


# speedup vs baseline: 3.4456x; 1.3297x over previous
import functools

import jax
import jax.numpy as jnp
from jax import lax
from jax.experimental import pallas as pl
from jax.experimental.pallas import tpu as pltpu
from jax.experimental.pallas import tpu_sc as plsc

EMBED_DIM = 64
SCALE = 8.0
LOWEST = 1e-13
T = 4


@jax.jit
def _emb_lookup(tokens, table):
    R, S = tokens.shape
    D = table.shape[1]
    info = plsc.get_sparse_core_info()
    NW = info.num_cores * info.num_subcores
    rows_per_w = R // NW
    n_chunks = rows_per_w // T
    assert n_chunks * T == rows_per_w and n_chunks % 2 == 0

    mesh = plsc.VectorSubcoreMesh(core_axis_name="c", subcore_axis_name="s")

    @functools.partial(
        pl.kernel,
        mesh=mesh,
        out_type=jax.ShapeDtypeStruct((R, S, 2 * D), jnp.float32),
        compiler_params=pltpu.CompilerParams(use_tc_tiling_on_sc=False),
        scratch_types=[
            pltpu.VMEM((2, T, S), jnp.int32),
            pltpu.VMEM((2, T, S, D), jnp.float32),
            pltpu.SemaphoreType.DMA((2,)),
        ],
    )
    def emb(tok_hbm, table_hbm, out_hbm, idx_v, rows_v, gsem):
        wid = lax.axis_index("s") * info.num_cores + lax.axis_index("c")
        base = wid * rows_per_w

        def stage(g, buf):
            row0 = base + g * T
            pltpu.sync_copy(tok_hbm.at[pl.ds(row0, T)], idx_v.at[buf])
            for t in range(T):
                pltpu.async_copy(
                    table_hbm.at[idx_v.at[buf, t, pl.ds(0, 128)]],
                    rows_v.at[buf, t, pl.ds(0, 128)],
                    gsem.at[buf],
                )
                pltpu.async_copy(
                    table_hbm.at[idx_v.at[buf, t, pl.ds(128, S - 128)]],
                    rows_v.at[buf, t, pl.ds(128, S - 128)],
                    gsem.at[buf],
                )

        def process(g, buf):
            pltpu.make_async_copy(
                out_hbm.at[pl.ds(0, T), :, pl.ds(0, D)],
                rows_v.at[buf],
                gsem.at[buf],
            ).wait()

            for t in range(T):

                def scale4(r4, _, t=t):
                    r = r4 * 4
                    for u in range(4):
                        for j in range(D // 16):
                            sl = pl.ds(j * 16, 16)
                            rows_v[buf, t, r + u, sl] = (
                                rows_v[buf, t, r + u, sl] * SCALE + LOWEST
                            )
                    return 0

                lax.fori_loop(0, S // 4, scale4, 0)
            pltpu.sync_copy(
                rows_v.at[buf],
                out_hbm.at[pl.ds(base + g * T, T), :, pl.ds(0, D)],
            )

        stage(0, 0)

        def pair_body(i, _):
            g0 = i * 2
            stage(g0 + 1, 1)
            process(g0, 0)

            @pl.when(g0 + 2 < n_chunks)
            def _():
                stage(g0 + 2, 0)

            process(g0 + 1, 1)
            return 0

        lax.fori_loop(0, n_chunks // 2, pair_body, 0)

    return emb(tokens, table)[..., :D]


def kernel(token_tensor, table):
    return _emb_lookup(token_tensor, table)
